# baseline (speedup 1.0000x reference)
import jax
import jax.numpy as jnp
from jax.experimental import pallas as pl
from jax.experimental.pallas import tpu as pltpu

_N = 10000
_RB = 400
_NB = _N // _RB


def _split_bf16(a):
    hi = a.astype(jnp.bfloat16)
    lo = (a - hi.astype(jnp.float32)).astype(jnp.bfloat16)
    return hi, lo


def _mk_s(x_ref, w1_ref, s_ref):
    xh, xl = _split_bf16(x_ref[...])
    wh, wl = _split_bf16(w1_ref[...])
    cross = (jnp.dot(xh, wl, preferred_element_type=jnp.float32)
             + jnp.dot(xl, wh, preferred_element_type=jnp.float32))
    s_ref[...] = jnp.dot(xh, wh, preferred_element_type=jnp.float32) + cross


def _stream_a(s_ref, b1_ref, wch_ref, wcl_ref, bcat_ref, adj_ref,
              res_ref):
    acc = jnp.dot(adj_ref[...], s_ref[...],
                  preferred_element_type=jnp.float32)
    h = jnp.maximum(acc + b1_ref[...], 0.0)
    hh, hl = _split_bf16(h)
    wch = wch_ref[...]
    wcl = wcl_ref[...]
    res = (jnp.dot(hh, wch, preferred_element_type=jnp.float32)
           + jnp.dot(hh, wcl, preferred_element_type=jnp.float32)
           + jnp.dot(hl, wch, preferred_element_type=jnp.float32))
    res_ref[...] = res + bcat_ref[...]


def _mk_al(alraw_ref, al_ref):
    alr = alraw_ref[...]
    e = jnp.exp(alr - jnp.max(alr))
    al_ref[...] = e / jnp.sum(e)


def _stream_b(m_ref, b2_ref, adj_ref, out1_ref):
    acc = jnp.dot(adj_ref[...], m_ref[...],
                  preferred_element_type=jnp.float32)
    x2 = acc + b2_ref[...]
    sh = x2 - jnp.max(x2, axis=1, keepdims=True)
    out1_ref[...] = sh - jnp.log(jnp.sum(jnp.exp(sh), axis=1, keepdims=True))


def kernel(x, adj, W1, b1, W2, b2, We, be, att):
    b1r = b1.reshape(1, 16)
    b2r = b2.reshape(1, 16)
    ber = be.reshape(1, 16)

    wcat = jnp.concatenate([att, We.T, W2], axis=1)
    wcat_h = wcat.astype(jnp.bfloat16)
    wcat_l = (wcat - wcat_h.astype(jnp.float32)).astype(jnp.bfloat16)

    s = pl.pallas_call(
        _mk_s,
        out_shape=jax.ShapeDtypeStruct((_N, 16), jnp.float32),
    )(x, W1)

    bcat = jnp.concatenate(
        [jnp.zeros((1, 16), jnp.float32), ber,
         jnp.zeros((1, 16), jnp.float32)], axis=1)

    res = pl.pallas_call(
        _stream_a,
        grid=(_NB,),
        in_specs=[
            pl.BlockSpec((_N, 16), lambda i: (0, 0)),
            pl.BlockSpec((1, 16), lambda i: (0, 0)),
            pl.BlockSpec((16, 48), lambda i: (0, 0)),
            pl.BlockSpec((16, 48), lambda i: (0, 0)),
            pl.BlockSpec((1, 48), lambda i: (0, 0)),
            pl.BlockSpec((_RB, _N), lambda i: (i, 0)),
        ],
        out_specs=pl.BlockSpec((_RB, 48), lambda i: (i, 0)),
        out_shape=jax.ShapeDtypeStruct((_N, 48), jnp.float32),
        compiler_params=pltpu.CompilerParams(
            dimension_semantics=("arbitrary",),
        ),
    )(s, b1r, wcat_h, wcat_l, bcat, adj)

    alraw = res[:, 0:16]
    y = res[:, 16:32]
    m = res[:, 32:48]

    alraw2 = alraw.reshape(1250, 128)

    al2 = pl.pallas_call(
        _mk_al,
        out_shape=jax.ShapeDtypeStruct((1250, 128), jnp.float32),
    )(alraw2)

    out1 = pl.pallas_call(
        _stream_b,
        grid=(_NB,),
        in_specs=[
            pl.BlockSpec((_N, 16), lambda i: (0, 0)),
            pl.BlockSpec((1, 16), lambda i: (0, 0)),
            pl.BlockSpec((_RB, _N), lambda i: (i, 0)),
        ],
        out_specs=pl.BlockSpec((_RB, 16), lambda i: (i, 0)),
        out_shape=jax.ShapeDtypeStruct((_N, 16), jnp.float32),
        compiler_params=pltpu.CompilerParams(
            dimension_semantics=("arbitrary",),
        ),
    )(m, b2r, adj)

    return out1, y, al2.reshape(_N, 16)


# speedup vs baseline: 1.0524x; 1.0524x over previous
import jax
import jax.numpy as jnp
from jax.experimental import pallas as pl
from jax.experimental.pallas import tpu as pltpu

_N = 10000
_RB = 400
_NB = _N // _RB


def _split_bf16(a):
    hi = a.astype(jnp.bfloat16)
    lo = (a - hi.astype(jnp.float32)).astype(jnp.bfloat16)
    return hi, lo


def _mk_s(x_ref, w1_ref, s_ref):
    s_ref[...] = jnp.dot(x_ref[...], w1_ref[...],
                         preferred_element_type=jnp.float32)


def _stream_a(s_ref, b1_ref, wc_ref, be_ref, adj_ref,
              alraw_ref, y_ref, m_ref):
    acc = jnp.dot(adj_ref[...], s_ref[...],
                  preferred_element_type=jnp.float32)
    h = jnp.maximum(acc + b1_ref[...], 0.0)
    res = jnp.dot(h, wc_ref[...], preferred_element_type=jnp.float32)
    alraw_ref[...] = res[:, 0:16]
    y_ref[...] = res[:, 16:32] + be_ref[...]
    m_ref[...] = res[:, 32:48]


def _mk_al(alraw_ref, al_ref):
    alr = alraw_ref[...]
    e = jnp.exp(alr - jnp.max(alr))
    al_ref[...] = e / jnp.sum(e)


def _stream_b(m_ref, b2_ref, adj_ref, out1_ref):
    acc = jnp.dot(adj_ref[...], m_ref[...],
                  preferred_element_type=jnp.float32)
    x2 = acc + b2_ref[...]
    sh = x2 - jnp.max(x2, axis=1, keepdims=True)
    out1_ref[...] = sh - jnp.log(jnp.sum(jnp.exp(sh), axis=1, keepdims=True))


def kernel(x, adj, W1, b1, W2, b2, We, be, att):
    b1r = b1.reshape(1, 16)
    b2r = b2.reshape(1, 16)
    ber = be.reshape(1, 16)

    wcat = jnp.concatenate([att, We.T, W2], axis=1)

    s = pl.pallas_call(
        _mk_s,
        out_shape=jax.ShapeDtypeStruct((_N, 16), jnp.float32),
    )(x, W1)

    alraw, y, m = pl.pallas_call(
        _stream_a,
        grid=(_NB,),
        in_specs=[
            pl.BlockSpec((_N, 16), lambda i: (0, 0)),
            pl.BlockSpec((1, 16), lambda i: (0, 0)),
            pl.BlockSpec((16, 48), lambda i: (0, 0)),
            pl.BlockSpec((1, 16), lambda i: (0, 0)),
            pl.BlockSpec((_RB, _N), lambda i: (i, 0)),
        ],
        out_specs=[
            pl.BlockSpec((_RB, 16), lambda i: (i, 0)),
            pl.BlockSpec((_RB, 16), lambda i: (i, 0)),
            pl.BlockSpec((_RB, 16), lambda i: (i, 0)),
        ],
        out_shape=[
            jax.ShapeDtypeStruct((_N, 16), jnp.float32),
            jax.ShapeDtypeStruct((_N, 16), jnp.float32),
            jax.ShapeDtypeStruct((_N, 16), jnp.float32),
        ],
        compiler_params=pltpu.CompilerParams(
            dimension_semantics=("arbitrary",),
        ),
    )(s, b1r, wcat, ber, adj)

    alraw2 = alraw.reshape(1250, 128)

    al2 = pl.pallas_call(
        _mk_al,
        out_shape=jax.ShapeDtypeStruct((1250, 128), jnp.float32),
    )(alraw2)

    out1 = pl.pallas_call(
        _stream_b,
        grid=(_NB,),
        in_specs=[
            pl.BlockSpec((_N, 16), lambda i: (0, 0)),
            pl.BlockSpec((1, 16), lambda i: (0, 0)),
            pl.BlockSpec((_RB, _N), lambda i: (i, 0)),
        ],
        out_specs=pl.BlockSpec((_RB, 16), lambda i: (i, 0)),
        out_shape=jax.ShapeDtypeStruct((_N, 16), jnp.float32),
        compiler_params=pltpu.CompilerParams(
            dimension_semantics=("arbitrary",),
        ),
    )(m, b2r, adj)

    return out1, y, al2.reshape(_N, 16)


# speedup vs baseline: 1.0833x; 1.0294x over previous
import jax
import jax.numpy as jnp
from jax.experimental import pallas as pl
from jax.experimental.pallas import tpu as pltpu

_N = 10000
_RB = 400
_NB = _N // _RB


def _stream_a(x_ref, w1_ref, b1_ref, wc_ref, be_ref, adj_ref,
              alraw_ref, y_ref, m_ref, s_ref):
    i = pl.program_id(0)

    @pl.when(i == 0)
    def _():
        s_ref[...] = jnp.dot(x_ref[...], w1_ref[...],
                             preferred_element_type=jnp.float32)

    acc = jnp.dot(adj_ref[...], s_ref[...],
                  preferred_element_type=jnp.float32)
    h = jnp.maximum(acc + b1_ref[...], 0.0)
    res = jnp.dot(h, wc_ref[...], preferred_element_type=jnp.float32)
    alraw_ref[...] = res[:, 0:16]
    y_ref[...] = res[:, 16:32] + be_ref[...]
    m_ref[...] = res[:, 32:48]


def _stream_b(alraw_ref, m_ref, b2_ref, adj_ref, out1_ref, al_ref):
    i = pl.program_id(0)

    @pl.when(i == 0)
    def _():
        alr = alraw_ref[...]
        e = jnp.exp(alr - jnp.max(alr))
        al_ref[...] = e / jnp.sum(e)

    acc = jnp.dot(adj_ref[...], m_ref[...],
                  preferred_element_type=jnp.float32)
    x2 = acc + b2_ref[...]
    sh = x2 - jnp.max(x2, axis=1, keepdims=True)
    out1_ref[...] = sh - jnp.log(jnp.sum(jnp.exp(sh), axis=1, keepdims=True))


def kernel(x, adj, W1, b1, W2, b2, We, be, att):
    b1r = b1.reshape(1, 16)
    b2r = b2.reshape(1, 16)
    ber = be.reshape(1, 16)

    wcat = jnp.concatenate([att, We.T, W2], axis=1)

    alraw, y, m = pl.pallas_call(
        _stream_a,
        grid=(_NB,),
        in_specs=[
            pl.BlockSpec((_N, 128), lambda i: (0, 0)),
            pl.BlockSpec((128, 16), lambda i: (0, 0)),
            pl.BlockSpec((1, 16), lambda i: (0, 0)),
            pl.BlockSpec((16, 48), lambda i: (0, 0)),
            pl.BlockSpec((1, 16), lambda i: (0, 0)),
            pl.BlockSpec((_RB, _N), lambda i: (i, 0)),
        ],
        out_specs=[
            pl.BlockSpec((_RB, 16), lambda i: (i, 0)),
            pl.BlockSpec((_RB, 16), lambda i: (i, 0)),
            pl.BlockSpec((_RB, 16), lambda i: (i, 0)),
        ],
        out_shape=[
            jax.ShapeDtypeStruct((_N, 16), jnp.float32),
            jax.ShapeDtypeStruct((_N, 16), jnp.float32),
            jax.ShapeDtypeStruct((_N, 16), jnp.float32),
        ],
        scratch_shapes=[pltpu.VMEM((_N, 16), jnp.float32)],
        compiler_params=pltpu.CompilerParams(
            dimension_semantics=("arbitrary",),
        ),
    )(x, W1, b1r, wcat, ber, adj)

    alraw2 = alraw.reshape(1250, 128)

    out1, al2 = pl.pallas_call(
        _stream_b,
        grid=(_NB,),
        in_specs=[
            pl.BlockSpec((1250, 128), lambda i: (0, 0)),
            pl.BlockSpec((_N, 16), lambda i: (0, 0)),
            pl.BlockSpec((1, 16), lambda i: (0, 0)),
            pl.BlockSpec((_RB, _N), lambda i: (i, 0)),
        ],
        out_specs=[
            pl.BlockSpec((_RB, 16), lambda i: (i, 0)),
            pl.BlockSpec((1250, 128), lambda i: (0, 0)),
        ],
        out_shape=[
            jax.ShapeDtypeStruct((_N, 16), jnp.float32),
            jax.ShapeDtypeStruct((1250, 128), jnp.float32),
        ],
        compiler_params=pltpu.CompilerParams(
            dimension_semantics=("arbitrary",),
        ),
    )(alraw2, m, b2r, adj)

    return out1, y, al2.reshape(_N, 16)
